# baseline (speedup 1.0000x reference)
import functools

import numpy as np
import jax
import jax.numpy as jnp
from jax import lax
from jax.experimental import pallas as pl
from jax.experimental.pallas import tpu as pltpu
from jax.experimental.pallas import tpu_sc as plsc

N_B = 2
LQ = 8192
DM = 256
NH = 8
NP = 4
H = 128
W = 128
LIN = H * W
HD = DM // NH

NC, NS = 2, 16
NW = NC * NS
NSLICE = 4
LQS = LQ // NSLICE
QPW = LQS // NS
CQ = 8
NCHUNK = QPW // CQ
KPQ = NH * NP * 4
CI = CQ * KPQ

_P = np.zeros((4, NH * NP, KPQ), np.float32)
_DVEC = np.zeros((1, KPQ), np.float32)
_HVEC = np.zeros((1, KPQ), np.float32)
_DOFF = (0.0, float(NH), float(NH * W), float(NH * (W + 1)))
for _h in range(NH):
    for _p in range(NP):
        for _j in range(4):
            _c = _h * 16 + _j * 4 + _p
            _P[_j, _h * 4 + _p, _c] = 1.0
            _DVEC[0, _c] = _DOFF[_j]
            _HVEC[0, _c] = _h
_G = np.kron(np.eye(NH, dtype=np.float32), np.ones((NP, NP), np.float32))


def _value_kernel(x_ref, wv_ref, bv_ref, m_ref, out_ref):
    v = jnp.dot(x_ref[0].astype(jnp.bfloat16),
                wv_ref[...].astype(jnp.bfloat16),
                preferred_element_type=jnp.float32)
    v = v + bv_ref[...]
    v = v * (1.0 - m_ref[0, 0][:, None])
    out_ref[...] = v[None]


def _value_tables(x, w_v, b_v1, maskf):
    bt = 2048
    nt = LIN // bt
    return pl.pallas_call(
        _value_kernel,
        grid=(N_B, nt),
        in_specs=[
            pl.BlockSpec((1, bt, DM), lambda b, t: (b, t, 0)),
            pl.BlockSpec((DM, DM), lambda b, t: (0, 0)),
            pl.BlockSpec((1, DM), lambda b, t: (0, 0)),
            pl.BlockSpec((1, 1, bt), lambda b, t: (b, 0, t)),
        ],
        out_specs=pl.BlockSpec((1, bt, DM), lambda b, t: (b, t, 0)),
        out_shape=jax.ShapeDtypeStruct((N_B, LIN, DM), jnp.float32),
    )(x, w_v, b_v1, maskf)


def _axis_terms(coord, extent):
    f0 = jnp.floor(coord)
    frac = coord - f0
    v0 = ((f0 >= 0.0) & (f0 <= extent - 1.0)).astype(jnp.float32)
    v1 = ((f0 >= -1.0) & (f0 <= extent - 2.0)).astype(jnp.float32)
    b = jnp.clip(f0, 0.0, extent - 2.0)
    c0 = jnp.clip(f0, 0.0, extent - 1.0)
    c1 = jnp.clip(f0 + 1.0, 0.0, extent - 1.0)
    w0 = (1.0 - frac) * v0
    w1 = frac * v1
    cw0 = w0 * (c0 == b) + w1 * (c1 == b)
    cw1 = w0 * (c0 == b + 1.0) + w1 * (c1 == b + 1.0)
    return b, cw0, cw1


def _idxw_kernel(q_ref, rp_ref, wox_ref, woy_ref, box_ref, boy_ref,
                 wa_ref, ba_ref, p0_ref, p1_ref, p2_ref, p3_ref, ps_ref,
                 dv_ref, hv_ref, g_ref, gidx_ref, gw_ref):
    b = pl.program_id(0)
    q = q_ref[0]
    offx = jnp.dot(q, wox_ref[...], preferred_element_type=jnp.float32) + box_ref[...]
    offy = jnp.dot(q, woy_ref[...], preferred_element_type=jnp.float32) + boy_ref[...]
    logits = jnp.dot(q, wa_ref[...], preferred_element_type=jnp.float32) + ba_ref[...]
    e = jnp.exp(logits)
    s = jnp.dot(e, g_ref[...], preferred_element_type=jnp.float32)
    attn = e / s

    rp = rp_ref[0]
    rx = rp[:, 0:1]
    ry = rp[:, 1:2]
    rw = rp[:, 2:3]
    rh = rp[:, 3:4]
    locx = rx + offx * 0.125 * rw
    locy = ry + offy * 0.125 * rh
    x = locx * W - 0.5
    y = locy * H - 0.5
    bx, cx0, cx1 = _axis_terms(x, float(W))
    by, cy0, cy1 = _axis_terms(y, float(H))
    base = by * W + bx

    w0 = attn * cy0 * cx0
    w1 = attn * cy0 * cx1
    w2 = attn * cy1 * cx0
    w3 = attn * cy1 * cx1
    hp = lax.Precision.HIGHEST
    gw = (jnp.dot(w0, p0_ref[...], preferred_element_type=jnp.float32)
          + jnp.dot(w1, p1_ref[...], preferred_element_type=jnp.float32)
          + jnp.dot(w2, p2_ref[...], preferred_element_type=jnp.float32)
          + jnp.dot(w3, p3_ref[...], preferred_element_type=jnp.float32))
    gb = jnp.dot(base, ps_ref[...], precision=hp,
                 preferred_element_type=jnp.float32) * float(NH)
    gb = gb + dv_ref[...] + hv_ref[...] + (b * (NH * LIN)).astype(jnp.float32)
    gidx_ref[...] = gb.astype(jnp.int32)[None]
    gw_ref[...] = gw[None]


def _index_weights(query, rp, woffx, woffy, boffx, boffy, w_attn, b_attn, k):
    qt = LQS
    nt = 1
    full = lambda shape: pl.BlockSpec(shape, lambda b, t, _s=shape: tuple(0 for _ in _s))
    return pl.pallas_call(
        _idxw_kernel,
        grid=(N_B, nt),
        in_specs=[
            pl.BlockSpec((1, qt, DM), lambda b, t: (b, k + t, 0)),
            pl.BlockSpec((1, qt, 4), lambda b, t: (b, k + t, 0)),
            full((DM, 32)), full((DM, 32)), full((1, 32)), full((1, 32)),
            full((DM, 32)), full((1, 32)),
            full((32, KPQ)), full((32, KPQ)), full((32, KPQ)), full((32, KPQ)),
            full((32, KPQ)), full((1, KPQ)), full((1, KPQ)), full((32, 32)),
        ],
        out_specs=[
            pl.BlockSpec((1, qt, KPQ), lambda b, t: (b, t, 0)),
            pl.BlockSpec((1, qt, KPQ), lambda b, t: (b, t, 0)),
        ],
        out_shape=[
            jax.ShapeDtypeStruct((N_B, LQS, KPQ), jnp.int32),
            jax.ShapeDtypeStruct((N_B, LQS, KPQ), jnp.float32),
        ],
    )(query, rp, woffx, woffy, boffx, boffy, w_attn, b_attn,
      jnp.asarray(_P[0]), jnp.asarray(_P[1]), jnp.asarray(_P[2]),
      jnp.asarray(_P[3]), jnp.asarray(_P.sum(0)), jnp.asarray(_DVEC),
      jnp.asarray(_HVEC), jnp.asarray(_G))


def _sc_body(table_hbm, gidx_hbm, gw_hbm, out_hbm,
             idx0_v, idx1_v, w0_v, w1_v, rows0_v, rows1_v, acc0_v, acc1_v,
             semg0, semg1, semi, semw, semo0, semo1):
    cid = lax.axis_index("c")
    sid = lax.axis_index("s")
    wid = cid * NS + sid
    b = wid // NS
    qs = wid % NS

    idx_b = (idx0_v, idx1_v)
    w_b = (w0_v, w1_v)
    rows_b = (rows0_v, rows1_v)
    acc_b = (acc0_v, acc1_v)
    semg_b = (semg0, semg1)
    semo_b = (semo0, semo1)

    pltpu.sync_copy(gidx_hbm.at[b, qs, 0], idx0_v)
    pltpu.sync_copy(gw_hbm.at[b, qs, 0], w0_v)
    pltpu.async_copy(table_hbm.at[idx0_v], rows0_v, semg0)
    pltpu.sync_copy(gidx_hbm.at[b, qs, 1], idx1_v)
    pltpu.sync_copy(gw_hbm.at[b, qs, 1], w1_v)
    pltpu.async_copy(table_hbm.at[idx1_v], rows1_v, semg1)

    def compute(rows_v, w_v, acc_v):
        def item(i, c2):
            rb = i * 16
            w16 = w_v[pl.ds(rb, 16)]
            a = [None] * 4
            c = [None] * 4
            for j in range(16):
                wj = w16[j]
                lo = wj * rows_v[rb + j, pl.ds(0, 16)]
                hi = wj * rows_v[rb + j, pl.ds(16, 16)]
                k = j & 3
                a[k] = lo if a[k] is None else a[k] + lo
                c[k] = hi if c[k] is None else c[k] + hi
            acc_v[pl.ds(i * HD, 16)] = (a[0] + a[1]) + (a[2] + a[3])
            acc_v[pl.ds(i * HD + 16, 16)] = (c[0] + c[1]) + (c[2] + c[3])
            return c2

        lax.fori_loop(0, CQ * NH, item, 0)

    def half(i, s):
        g = i * 2 + s
        more = i < NCHUNK // 2 - 1
        pltpu.make_async_copy(table_hbm.at[idx_b[s]], rows_b[s],
                              semg_b[s]).wait()

        @pl.when(more)
        def _():
            pltpu.async_copy(gidx_hbm.at[b, qs, g + 2], idx_b[s], semi)

        @pl.when(i >= 1)
        def _():
            pltpu.make_async_copy(
                acc_b[s], out_hbm.at[b, qs * NCHUNK + g - 2], semo_b[s]).wait()

        compute(rows_b[s], w_b[s], acc_b[s])

        @pl.when(more)
        def _():
            pltpu.async_copy(gw_hbm.at[b, qs, g + 2], w_b[s], semw)

        pltpu.async_copy(acc_b[s], out_hbm.at[b, qs * NCHUNK + g], semo_b[s])

        @pl.when(more)
        def _():
            pltpu.make_async_copy(gidx_hbm.at[b, qs, g + 2], idx_b[s],
                                  semi).wait()
            pltpu.make_async_copy(gw_hbm.at[b, qs, g + 2], w_b[s],
                                  semw).wait()
            pltpu.async_copy(table_hbm.at[idx_b[s]], rows_b[s], semg_b[s])

    def pair(i, carry):
        half(i, 0)
        half(i, 1)
        return carry

    lax.fori_loop(0, NCHUNK // 2, pair, 0)

    pltpu.make_async_copy(
        acc0_v, out_hbm.at[b, qs * NCHUNK + NCHUNK - 2], semo0).wait()
    pltpu.make_async_copy(
        acc1_v, out_hbm.at[b, qs * NCHUNK + NCHUNK - 1], semo1).wait()


def _sc_gather(table, gidx, gw):
    mesh = plsc.VectorSubcoreMesh(
        core_axis_name="c", subcore_axis_name="s",
        num_cores=NC, num_subcores=NS)
    f = pl.kernel(
        _sc_body,
        out_type=jax.ShapeDtypeStruct((N_B, NS * NCHUNK, CQ * DM), jnp.float32),
        mesh=mesh,
        scratch_types=[
            pltpu.VMEM((CI,), jnp.int32),
            pltpu.VMEM((CI,), jnp.int32),
            pltpu.VMEM((CI,), jnp.float32),
            pltpu.VMEM((CI,), jnp.float32),
            pltpu.VMEM((CI, HD), jnp.float32),
            pltpu.VMEM((CI, HD), jnp.float32),
            pltpu.VMEM((CQ * DM,), jnp.float32),
            pltpu.VMEM((CQ * DM,), jnp.float32),
            pltpu.SemaphoreType.DMA,
            pltpu.SemaphoreType.DMA,
            pltpu.SemaphoreType.DMA,
            pltpu.SemaphoreType.DMA,
            pltpu.SemaphoreType.DMA,
            pltpu.SemaphoreType.DMA,
        ],
        compiler_params=pltpu.CompilerParams(use_tc_tiling_on_sc=False),
    )
    return f(table, gidx, gw)


def _proj_kernel(x_ref, wo_ref, bo_ref, dest_ref, out_ref):
    del dest_ref
    out_ref[...] = (jnp.dot(x_ref[...].astype(jnp.bfloat16),
                            wo_ref[...].astype(jnp.bfloat16),
                            preferred_element_type=jnp.float32)
                    + bo_ref[...])


def _out_proj(sampled2d, w_o, b_o1, dest, k):
    bt = 1024
    nt = LQS // bt
    kb = LQS // bt * k
    lb = LQ // bt
    return pl.pallas_call(
        _proj_kernel,
        grid=(N_B, nt),
        in_specs=[
            pl.BlockSpec((bt, DM), lambda b, t: (b * nt + t, 0)),
            pl.BlockSpec((DM, DM), lambda b, t: (0, 0)),
            pl.BlockSpec((1, DM), lambda b, t: (0, 0)),
            pl.BlockSpec(memory_space=pl.ANY),
        ],
        out_specs=pl.BlockSpec((bt, DM), lambda b, t: (b * lb + kb + t, 0)),
        out_shape=jax.ShapeDtypeStruct((N_B * LQ, DM), jnp.float32),
        input_output_aliases={3: 0},
    )(sampled2d, w_o, b_o1, dest)


def kernel(query, reference_points, input_flatten, input_spatial_shapes,
           input_level_start_index, input_padding_mask, W_v, b_v, W_off,
           b_off, W_attn, b_attn, W_o, b_o):
    maskf = input_padding_mask.astype(jnp.float32).reshape(N_B, 1, LIN)
    value = _value_tables(input_flatten, W_v, b_v.reshape(1, DM), maskf)
    table = value.reshape(N_B * LIN * NH, HD)

    woffx = W_off[:, 0::2]
    woffy = W_off[:, 1::2]
    boffx = b_off[0::2].reshape(1, 32)
    boffy = b_off[1::2].reshape(1, 32)
    rp = reference_points.reshape(N_B, LQ, 4)
    b_attn1 = b_attn.reshape(1, 32)
    b_o1 = b_o.reshape(1, DM)

    dest = jnp.zeros((N_B * LQ, DM), jnp.float32)
    for k in range(NSLICE):
        gidx, gw = _index_weights(query, rp, woffx, woffy, boffx, boffy,
                                  W_attn, b_attn1, k)
        sampled = _sc_gather(table,
                             gidx.reshape(N_B, NS, NCHUNK, CI),
                             gw.reshape(N_B, NS, NCHUNK, CI))
        dest = _out_proj(sampled.reshape(N_B * LQS, DM), W_o, b_o1, dest, k)
    return dest.reshape(N_B, LQ, DM)


# speedup vs baseline: 1.0529x; 1.0529x over previous
import functools

import numpy as np
import jax
import jax.numpy as jnp
from jax import lax
from jax.experimental import pallas as pl
from jax.experimental.pallas import tpu as pltpu
from jax.experimental.pallas import tpu_sc as plsc

N_B = 2
LQ = 8192
DM = 256
NH = 8
NP = 4
H = 128
W = 128
LIN = H * W
HD = DM // NH

NC, NS = 2, 16
NW = NC * NS
NSLICE = 4
LQS = LQ // NSLICE
QPW = LQS // NS
CQ = 8
NCHUNK = QPW // CQ
KPQ = NH * NP * 4
CI = CQ * KPQ

_P = np.zeros((4, NH * NP, KPQ), np.float32)
_DVEC = np.zeros((1, KPQ), np.float32)
_HVEC = np.zeros((1, KPQ), np.float32)
_DOFF = (0.0, float(NH), float(NH * W), float(NH * (W + 1)))
for _h in range(NH):
    for _p in range(NP):
        for _j in range(4):
            _c = _h * 16 + _j * 4 + _p
            _P[_j, _h * 4 + _p, _c] = 1.0
            _DVEC[0, _c] = _DOFF[_j]
            _HVEC[0, _c] = _h
_G = np.kron(np.eye(NH, dtype=np.float32), np.ones((NP, NP), np.float32))


def _value_kernel(x_ref, wv_ref, bv_ref, m_ref, out_ref):
    v = jnp.dot(x_ref[0].astype(jnp.bfloat16),
                wv_ref[...].astype(jnp.bfloat16),
                preferred_element_type=jnp.float32)
    v = v + bv_ref[...]
    v = v * (1.0 - m_ref[0, 0][:, None])
    out_ref[...] = v.reshape(-1, 128)[None]


def _value_tables(x, w_v, b_v1, maskf):
    bt = 2048
    nt = LIN // bt
    return pl.pallas_call(
        _value_kernel,
        grid=(N_B, nt),
        in_specs=[
            pl.BlockSpec((1, bt, DM), lambda b, t: (b, t, 0)),
            pl.BlockSpec((DM, DM), lambda b, t: (0, 0)),
            pl.BlockSpec((1, DM), lambda b, t: (0, 0)),
            pl.BlockSpec((1, 1, bt), lambda b, t: (b, 0, t)),
        ],
        out_specs=pl.BlockSpec((1, 2 * bt, 128), lambda b, t: (b, t, 0)),
        out_shape=jax.ShapeDtypeStruct((N_B, 2 * LIN, 128), jnp.float32),
    )(x, w_v, b_v1, maskf)


def _axis_terms(coord, extent):
    f0 = jnp.floor(coord)
    frac = coord - f0
    v0 = ((f0 >= 0.0) & (f0 <= extent - 1.0)).astype(jnp.float32)
    v1 = ((f0 >= -1.0) & (f0 <= extent - 2.0)).astype(jnp.float32)
    b = jnp.clip(f0, 0.0, extent - 2.0)
    c0 = jnp.clip(f0, 0.0, extent - 1.0)
    c1 = jnp.clip(f0 + 1.0, 0.0, extent - 1.0)
    w0 = (1.0 - frac) * v0
    w1 = frac * v1
    cw0 = w0 * (c0 == b) + w1 * (c1 == b)
    cw1 = w0 * (c0 == b + 1.0) + w1 * (c1 == b + 1.0)
    return b, cw0, cw1


def _idxw_kernel(q_ref, rp_ref, wox_ref, woy_ref, box_ref, boy_ref,
                 wa_ref, ba_ref, p0_ref, p1_ref, p2_ref, p3_ref, ps_ref,
                 dv_ref, hv_ref, g_ref, gidx_ref, gw_ref):
    b = pl.program_id(0)
    q = q_ref[0]
    offx = jnp.dot(q, wox_ref[...], preferred_element_type=jnp.float32) + box_ref[...]
    offy = jnp.dot(q, woy_ref[...], preferred_element_type=jnp.float32) + boy_ref[...]
    logits = jnp.dot(q, wa_ref[...], preferred_element_type=jnp.float32) + ba_ref[...]
    e = jnp.exp(logits)
    s = jnp.dot(e, g_ref[...], preferred_element_type=jnp.float32)
    attn = e / s

    rp = rp_ref[0]
    rx = rp[:, 0:1]
    ry = rp[:, 1:2]
    rw = rp[:, 2:3]
    rh = rp[:, 3:4]
    locx = rx + offx * 0.125 * rw
    locy = ry + offy * 0.125 * rh
    x = locx * W - 0.5
    y = locy * H - 0.5
    bx, cx0, cx1 = _axis_terms(x, float(W))
    by, cy0, cy1 = _axis_terms(y, float(H))
    base = by * W + bx

    w0 = attn * cy0 * cx0
    w1 = attn * cy0 * cx1
    w2 = attn * cy1 * cx0
    w3 = attn * cy1 * cx1
    hp = lax.Precision.HIGHEST
    gw = (jnp.dot(w0, p0_ref[...], preferred_element_type=jnp.float32)
          + jnp.dot(w1, p1_ref[...], preferred_element_type=jnp.float32)
          + jnp.dot(w2, p2_ref[...], preferred_element_type=jnp.float32)
          + jnp.dot(w3, p3_ref[...], preferred_element_type=jnp.float32))
    gb = jnp.dot(base, ps_ref[...], precision=hp,
                 preferred_element_type=jnp.float32) * float(NH)
    gb = gb + dv_ref[...] + hv_ref[...] + (b * (NH * LIN)).astype(jnp.float32)
    gidx_ref[...] = gb.astype(jnp.int32)[None]
    gw_ref[...] = gw[None]


def _index_weights(query, rp, woffx, woffy, boffx, boffy, w_attn, b_attn, k):
    qt = LQS
    nt = 1
    full = lambda shape: pl.BlockSpec(shape, lambda b, t, _s=shape: tuple(0 for _ in _s))
    return pl.pallas_call(
        _idxw_kernel,
        grid=(N_B, nt),
        in_specs=[
            pl.BlockSpec((1, qt, DM), lambda b, t: (b, k + t, 0)),
            pl.BlockSpec((1, qt, 4), lambda b, t: (b, k + t, 0)),
            full((DM, 32)), full((DM, 32)), full((1, 32)), full((1, 32)),
            full((DM, 32)), full((1, 32)),
            full((32, KPQ)), full((32, KPQ)), full((32, KPQ)), full((32, KPQ)),
            full((32, KPQ)), full((1, KPQ)), full((1, KPQ)), full((32, 32)),
        ],
        out_specs=[
            pl.BlockSpec((1, qt, KPQ), lambda b, t: (b, t, 0)),
            pl.BlockSpec((1, qt, KPQ), lambda b, t: (b, t, 0)),
        ],
        out_shape=[
            jax.ShapeDtypeStruct((N_B, LQS, KPQ), jnp.int32),
            jax.ShapeDtypeStruct((N_B, LQS, KPQ), jnp.float32),
        ],
    )(query, rp, woffx, woffy, boffx, boffy, w_attn, b_attn,
      jnp.asarray(_P[0]), jnp.asarray(_P[1]), jnp.asarray(_P[2]),
      jnp.asarray(_P[3]), jnp.asarray(_P.sum(0)), jnp.asarray(_DVEC),
      jnp.asarray(_HVEC), jnp.asarray(_G))


def _sc_body(table_hbm, gidx_hbm, gw_hbm, out_hbm,
             idx0_v, idx1_v, w0_v, w1_v, rows0_v, rows1_v, acc0_v, acc1_v,
             semg0, semg1, semi, semw, semo0, semo1):
    cid = lax.axis_index("c")
    sid = lax.axis_index("s")
    wid = cid * NS + sid
    b = wid // NS
    qs = wid % NS

    idx_b = (idx0_v, idx1_v)
    w_b = (w0_v, w1_v)
    rows_b = (rows0_v, rows1_v)
    acc_b = (acc0_v, acc1_v)
    semg_b = (semg0, semg1)
    semo_b = (semo0, semo1)

    pltpu.sync_copy(gidx_hbm.at[b, qs, 0], idx0_v)
    pltpu.sync_copy(gw_hbm.at[b, qs, 0], w0_v)
    pltpu.async_copy(table_hbm.at[idx0_v], rows0_v, semg0)
    pltpu.sync_copy(gidx_hbm.at[b, qs, 1], idx1_v)
    pltpu.sync_copy(gw_hbm.at[b, qs, 1], w1_v)
    pltpu.async_copy(table_hbm.at[idx1_v], rows1_v, semg1)

    def compute(rows_v, w_v, acc_v):
        def item(i, c2):
            rb = i * 16
            w16 = w_v[pl.ds(rb, 16)]
            a = [None] * 4
            c = [None] * 4
            for j in range(16):
                wj = w16[j]
                lo = wj * rows_v[rb + j, pl.ds(0, 16)]
                hi = wj * rows_v[rb + j, pl.ds(16, 16)]
                k = j & 3
                a[k] = lo if a[k] is None else a[k] + lo
                c[k] = hi if c[k] is None else c[k] + hi
            acc_v[pl.ds(i * HD, 16)] = (a[0] + a[1]) + (a[2] + a[3])
            acc_v[pl.ds(i * HD + 16, 16)] = (c[0] + c[1]) + (c[2] + c[3])
            return c2

        lax.fori_loop(0, CQ * NH, item, 0)

    def half(i, s):
        g = i * 2 + s
        more = i < NCHUNK // 2 - 1
        pltpu.make_async_copy(table_hbm.at[idx_b[s]], rows_b[s],
                              semg_b[s]).wait()

        @pl.when(more)
        def _():
            pltpu.async_copy(gidx_hbm.at[b, qs, g + 2], idx_b[s], semi)

        @pl.when(i >= 1)
        def _():
            pltpu.make_async_copy(
                acc_b[s], out_hbm.at[b, qs * NCHUNK + g - 2], semo_b[s]).wait()

        compute(rows_b[s], w_b[s], acc_b[s])

        @pl.when(more)
        def _():
            pltpu.async_copy(gw_hbm.at[b, qs, g + 2], w_b[s], semw)

        pltpu.async_copy(acc_b[s], out_hbm.at[b, qs * NCHUNK + g], semo_b[s])

        @pl.when(more)
        def _():
            pltpu.make_async_copy(gidx_hbm.at[b, qs, g + 2], idx_b[s],
                                  semi).wait()
            pltpu.make_async_copy(gw_hbm.at[b, qs, g + 2], w_b[s],
                                  semw).wait()
            pltpu.async_copy(table_hbm.at[idx_b[s]], rows_b[s], semg_b[s])

    def pair(i, carry):
        half(i, 0)
        half(i, 1)
        return carry

    lax.fori_loop(0, NCHUNK // 2, pair, 0)

    pltpu.make_async_copy(
        acc0_v, out_hbm.at[b, qs * NCHUNK + NCHUNK - 2], semo0).wait()
    pltpu.make_async_copy(
        acc1_v, out_hbm.at[b, qs * NCHUNK + NCHUNK - 1], semo1).wait()


def _sc_gather(table, gidx, gw):
    mesh = plsc.VectorSubcoreMesh(
        core_axis_name="c", subcore_axis_name="s",
        num_cores=NC, num_subcores=NS)
    f = pl.kernel(
        _sc_body,
        out_type=jax.ShapeDtypeStruct((N_B, NS * NCHUNK, CQ * DM), jnp.float32),
        mesh=mesh,
        scratch_types=[
            pltpu.VMEM((CI,), jnp.int32),
            pltpu.VMEM((CI,), jnp.int32),
            pltpu.VMEM((CI,), jnp.float32),
            pltpu.VMEM((CI,), jnp.float32),
            pltpu.VMEM((CI, HD), jnp.float32),
            pltpu.VMEM((CI, HD), jnp.float32),
            pltpu.VMEM((CQ * DM,), jnp.float32),
            pltpu.VMEM((CQ * DM,), jnp.float32),
            pltpu.SemaphoreType.DMA,
            pltpu.SemaphoreType.DMA,
            pltpu.SemaphoreType.DMA,
            pltpu.SemaphoreType.DMA,
            pltpu.SemaphoreType.DMA,
            pltpu.SemaphoreType.DMA,
        ],
        compiler_params=pltpu.CompilerParams(use_tc_tiling_on_sc=False),
    )
    return f(table, gidx, gw)


def _proj_body(x_ref, wo_ref, bo_ref, out_ref):
    out_ref[...] = (jnp.dot(x_ref[...].astype(jnp.bfloat16),
                            wo_ref[...].astype(jnp.bfloat16),
                            preferred_element_type=jnp.float32)
                    + bo_ref[...])


def _proj_kernel(x_ref, wo_ref, bo_ref, out_ref):
    _proj_body(x_ref, wo_ref, bo_ref, out_ref)


def _proj_kernel_alias(x_ref, wo_ref, bo_ref, dest_ref, out_ref):
    del dest_ref
    _proj_body(x_ref, wo_ref, bo_ref, out_ref)


def _out_proj(sampled2d, w_o, b_o1, dest, k):
    bt = 1024
    nt = LQS // bt
    kb = LQS // bt * k
    lb = LQ // bt
    in_specs = [
        pl.BlockSpec((bt, DM), lambda b, t: (b * nt + t, 0)),
        pl.BlockSpec((DM, DM), lambda b, t: (0, 0)),
        pl.BlockSpec((1, DM), lambda b, t: (0, 0)),
    ]
    args = (sampled2d, w_o, b_o1)
    body = _proj_kernel
    aliases = {}
    if dest is not None:
        in_specs.append(pl.BlockSpec(memory_space=pl.ANY))
        args = args + (dest,)
        body = _proj_kernel_alias
        aliases = {3: 0}
    return pl.pallas_call(
        body,
        grid=(N_B, nt),
        in_specs=in_specs,
        out_specs=pl.BlockSpec((bt, DM), lambda b, t: (b * lb + kb + t, 0)),
        out_shape=jax.ShapeDtypeStruct((N_B * LQ, DM), jnp.float32),
        input_output_aliases=aliases,
    )(*args)


def kernel(query, reference_points, input_flatten, input_spatial_shapes,
           input_level_start_index, input_padding_mask, W_v, b_v, W_off,
           b_off, W_attn, b_attn, W_o, b_o):
    maskf = input_padding_mask.astype(jnp.float32).reshape(N_B, 1, LIN)
    value = _value_tables(input_flatten, W_v, b_v.reshape(1, DM), maskf)
    table = value.reshape(N_B * LIN * NH, HD)

    woffx = W_off[:, 0::2]
    woffy = W_off[:, 1::2]
    boffx = b_off[0::2].reshape(1, 32)
    boffy = b_off[1::2].reshape(1, 32)
    rp = reference_points.reshape(N_B, LQ, 4)
    b_attn1 = b_attn.reshape(1, 32)
    b_o1 = b_o.reshape(1, DM)

    dest = None
    for k in range(NSLICE):
        gidx, gw = _index_weights(query, rp, woffx, woffy, boffx, boffy,
                                  W_attn, b_attn1, k)
        sampled = _sc_gather(table,
                             gidx.reshape(N_B, NS, NCHUNK, CI),
                             gw.reshape(N_B, NS, NCHUNK, CI))
        dest = _out_proj(sampled.reshape(N_B * LQS, DM), W_o, b_o1, dest, k)
    return dest.reshape(N_B, LQ, DM)


# speedup vs baseline: 1.1216x; 1.0652x over previous
import functools

import numpy as np
import jax
import jax.numpy as jnp
from jax import lax
from jax.experimental import pallas as pl
from jax.experimental.pallas import tpu as pltpu
from jax.experimental.pallas import tpu_sc as plsc

N_B = 2
LQ = 8192
DM = 256
NH = 8
NP = 4
H = 128
W = 128
LIN = H * W
HD = DM // NH

NC, NS = 2, 16
NW = NC * NS
NSLICE = 4
LQS = LQ // NSLICE
QPW = LQS // NS
CQ = 8
NCHUNK = QPW // CQ
KPQ = NH * NP * 4
CI = CQ * KPQ

_P = np.zeros((4, NH * NP, KPQ), np.float32)
_DVEC = np.zeros((1, KPQ), np.float32)
_HVEC = np.zeros((1, KPQ), np.float32)
_DOFF = (0.0, float(NH), float(NH * W), float(NH * (W + 1)))
for _h in range(NH):
    for _p in range(NP):
        for _j in range(4):
            _c = _h * 16 + _j * 4 + _p
            _P[_j, _h * 4 + _p, _c] = 1.0
            _DVEC[0, _c] = _DOFF[_j]
            _HVEC[0, _c] = _h
_G = np.kron(np.eye(NH, dtype=np.float32), np.ones((NP, NP), np.float32))


def _value_kernel(x_ref, wv_ref, bv_ref, m_ref, out_ref):
    v = jnp.dot(x_ref[0].astype(jnp.bfloat16),
                wv_ref[...].astype(jnp.bfloat16),
                preferred_element_type=jnp.float32)
    v = v + bv_ref[...]
    v = v * (1.0 - m_ref[0, 0][:, None])
    out_ref[...] = v.reshape(-1, 128)[None]


def _value_tables(x, w_v, b_v1, maskf):
    bt = 2048
    nt = LIN // bt
    return pl.pallas_call(
        _value_kernel,
        grid=(N_B, nt),
        in_specs=[
            pl.BlockSpec((1, bt, DM), lambda b, t: (b, t, 0)),
            pl.BlockSpec((DM, DM), lambda b, t: (0, 0)),
            pl.BlockSpec((1, DM), lambda b, t: (0, 0)),
            pl.BlockSpec((1, 1, bt), lambda b, t: (b, 0, t)),
        ],
        out_specs=pl.BlockSpec((1, 2 * bt, 128), lambda b, t: (b, t, 0)),
        out_shape=jax.ShapeDtypeStruct((N_B, 2 * LIN, 128), jnp.float32),
    )(x, w_v, b_v1, maskf)


def _axis_terms(coord, extent):
    f0 = jnp.floor(coord)
    frac = coord - f0
    v0 = ((f0 >= 0.0) & (f0 <= extent - 1.0)).astype(jnp.float32)
    v1 = ((f0 >= -1.0) & (f0 <= extent - 2.0)).astype(jnp.float32)
    b = jnp.clip(f0, 0.0, extent - 2.0)
    c0 = jnp.clip(f0, 0.0, extent - 1.0)
    c1 = jnp.clip(f0 + 1.0, 0.0, extent - 1.0)
    w0 = (1.0 - frac) * v0
    w1 = frac * v1
    cw0 = w0 * (c0 == b) + w1 * (c1 == b)
    cw1 = w0 * (c0 == b + 1.0) + w1 * (c1 == b + 1.0)
    return b, cw0, cw1


def _idxw_kernel(q_ref, rp_ref, wox_ref, woy_ref, box_ref, boy_ref,
                 wa_ref, ba_ref, p0_ref, p1_ref, p2_ref, p3_ref, ps_ref,
                 dv_ref, hv_ref, g_ref, gidx_ref, gw_ref):
    b = pl.program_id(0)
    q = q_ref[0]
    offx = jnp.dot(q, wox_ref[...], preferred_element_type=jnp.float32) + box_ref[...]
    offy = jnp.dot(q, woy_ref[...], preferred_element_type=jnp.float32) + boy_ref[...]
    logits = jnp.dot(q, wa_ref[...], preferred_element_type=jnp.float32) + ba_ref[...]
    e = jnp.exp(logits)
    s = jnp.dot(e, g_ref[...], preferred_element_type=jnp.float32)
    attn = e / s

    rp = rp_ref[0]
    rx = rp[:, 0:1]
    ry = rp[:, 1:2]
    rw = rp[:, 2:3]
    rh = rp[:, 3:4]
    locx = rx + offx * 0.125 * rw
    locy = ry + offy * 0.125 * rh
    x = locx * W - 0.5
    y = locy * H - 0.5
    bx, cx0, cx1 = _axis_terms(x, float(W))
    by, cy0, cy1 = _axis_terms(y, float(H))
    base = by * W + bx

    w0 = attn * cy0 * cx0
    w1 = attn * cy0 * cx1
    w2 = attn * cy1 * cx0
    w3 = attn * cy1 * cx1
    hp = lax.Precision.HIGHEST
    gw = (jnp.dot(w0, p0_ref[...], preferred_element_type=jnp.float32)
          + jnp.dot(w1, p1_ref[...], preferred_element_type=jnp.float32)
          + jnp.dot(w2, p2_ref[...], preferred_element_type=jnp.float32)
          + jnp.dot(w3, p3_ref[...], preferred_element_type=jnp.float32))
    gb = jnp.dot(base, ps_ref[...], precision=hp,
                 preferred_element_type=jnp.float32) * float(NH)
    gb = gb + dv_ref[...] + hv_ref[...] + (b * (NH * LIN)).astype(jnp.float32)
    gidx_ref[...] = gb.astype(jnp.int32)[None]
    gw_ref[...] = gw[None]


def _index_weights(query, rp, woffx, woffy, boffx, boffy, w_attn, b_attn, k):
    qt = LQS
    nt = 1
    full = lambda shape: pl.BlockSpec(shape, lambda b, t, _s=shape: tuple(0 for _ in _s))
    return pl.pallas_call(
        _idxw_kernel,
        grid=(N_B, nt),
        in_specs=[
            pl.BlockSpec((1, qt, DM), lambda b, t: (b, k + t, 0)),
            pl.BlockSpec((1, qt, 4), lambda b, t: (b, k + t, 0)),
            full((DM, 32)), full((DM, 32)), full((1, 32)), full((1, 32)),
            full((DM, 32)), full((1, 32)),
            full((32, KPQ)), full((32, KPQ)), full((32, KPQ)), full((32, KPQ)),
            full((32, KPQ)), full((1, KPQ)), full((1, KPQ)), full((32, 32)),
        ],
        out_specs=[
            pl.BlockSpec((1, qt, KPQ), lambda b, t: (b, t, 0)),
            pl.BlockSpec((1, qt, KPQ), lambda b, t: (b, t, 0)),
        ],
        out_shape=[
            jax.ShapeDtypeStruct((N_B, LQS, KPQ), jnp.int32),
            jax.ShapeDtypeStruct((N_B, LQS, KPQ), jnp.float32),
        ],
    )(query, rp, woffx, woffy, boffx, boffy, w_attn, b_attn,
      jnp.asarray(_P[0]), jnp.asarray(_P[1]), jnp.asarray(_P[2]),
      jnp.asarray(_P[3]), jnp.asarray(_P.sum(0)), jnp.asarray(_DVEC),
      jnp.asarray(_HVEC), jnp.asarray(_G))


def _sc_body(table_hbm, gidx_hbm, gw_hbm, out_hbm,
             idx0_v, idx1_v, w0_v, w1_v, rows0_v, rows1_v, acc0_v, acc1_v,
             semg0, semg1, semi, semw, semo0, semo1):
    cid = lax.axis_index("c")
    sid = lax.axis_index("s")
    wid = cid * NS + sid
    b = wid // NS
    qs = wid % NS

    idx_b = (idx0_v, idx1_v)
    w_b = (w0_v, w1_v)
    rows_b = (rows0_v, rows1_v)
    acc_b = (acc0_v, acc1_v)
    semg_b = (semg0, semg1)
    semo_b = (semo0, semo1)

    iw_base = (b * NS + qs) * (NCHUNK * CI)
    orow = (qs * NCHUNK) * (CQ * DM // 128)
    ORPC = CQ * DM // 128

    pltpu.sync_copy(gidx_hbm.at[pl.ds(iw_base, CI)], idx0_v)
    pltpu.sync_copy(gw_hbm.at[pl.ds(iw_base, CI)], w0_v)
    pltpu.async_copy(table_hbm.at[idx0_v], rows0_v, semg0)
    pltpu.sync_copy(gidx_hbm.at[pl.ds(iw_base + CI, CI)], idx1_v)
    pltpu.sync_copy(gw_hbm.at[pl.ds(iw_base + CI, CI)], w1_v)
    pltpu.async_copy(table_hbm.at[idx1_v], rows1_v, semg1)

    def compute(rows_v, w_v, acc_v):
        def item(q, c2):
            for h in range(NH):
                i = q * NH + h
                rb = i * 16
                w16 = w_v[pl.ds(rb, 16)]
                a = [None] * 4
                c = [None] * 4
                for j in range(16):
                    wj = w16[j]
                    lo = wj * rows_v[rb + j, pl.ds(0, 16)]
                    hi = wj * rows_v[rb + j, pl.ds(16, 16)]
                    k = j & 3
                    a[k] = lo if a[k] is None else a[k] + lo
                    c[k] = hi if c[k] is None else c[k] + hi
                row = q * 2 + h // 4
                col = (h % 4) * 32
                acc_v[row, pl.ds(col, 16)] = (a[0] + a[1]) + (a[2] + a[3])
                acc_v[row, pl.ds(col + 16, 16)] = (c[0] + c[1]) + (c[2] + c[3])
            return c2

        lax.fori_loop(0, CQ, item, 0)

    def half(i, s):
        g = i * 2 + s
        more = i < NCHUNK // 2 - 1
        pltpu.make_async_copy(table_hbm.at[idx_b[s]], rows_b[s],
                              semg_b[s]).wait()

        @pl.when(more)
        def _():
            pltpu.async_copy(gidx_hbm.at[pl.ds(iw_base + (g + 2) * CI, CI)],
                             idx_b[s], semi)

        @pl.when(i >= 1)
        def _():
            pltpu.make_async_copy(
                acc_b[s],
                out_hbm.at[b, pl.ds((orow + (g - 2) * ORPC), ORPC)],
                semo_b[s]).wait()

        compute(rows_b[s], w_b[s], acc_b[s])

        @pl.when(more)
        def _():
            pltpu.async_copy(gw_hbm.at[pl.ds(iw_base + (g + 2) * CI, CI)],
                             w_b[s], semw)

        pltpu.async_copy(acc_b[s],
                         out_hbm.at[b, pl.ds(orow + g * ORPC, ORPC)],
                         semo_b[s])

        @pl.when(more)
        def _():
            pltpu.make_async_copy(gidx_hbm.at[pl.ds(iw_base + (g + 2) * CI,
                                                    CI)],
                                  idx_b[s], semi).wait()
            pltpu.make_async_copy(gw_hbm.at[pl.ds(iw_base + (g + 2) * CI,
                                                  CI)],
                                  w_b[s], semw).wait()
            pltpu.async_copy(table_hbm.at[idx_b[s]], rows_b[s], semg_b[s])

    def pair(i, carry):
        half(i, 0)
        half(i, 1)
        return carry

    lax.fori_loop(0, NCHUNK // 2, pair, 0)

    pltpu.make_async_copy(
        acc0_v, out_hbm.at[b, pl.ds(orow + (NCHUNK - 2) * ORPC, ORPC)],
        semo0).wait()
    pltpu.make_async_copy(
        acc1_v, out_hbm.at[b, pl.ds(orow + (NCHUNK - 1) * ORPC, ORPC)],
        semo1).wait()


def _sc_gather(table, gidx, gw):
    mesh = plsc.VectorSubcoreMesh(
        core_axis_name="c", subcore_axis_name="s",
        num_cores=NC, num_subcores=NS)
    f = pl.kernel(
        _sc_body,
        out_type=jax.ShapeDtypeStruct(
            (N_B, NS * NCHUNK * (CQ * DM // 128), 128), jnp.float32),
        mesh=mesh,
        scratch_types=[
            pltpu.VMEM((CI,), jnp.int32),
            pltpu.VMEM((CI,), jnp.int32),
            pltpu.VMEM((CI,), jnp.float32),
            pltpu.VMEM((CI,), jnp.float32),
            pltpu.VMEM((CI, HD), jnp.float32),
            pltpu.VMEM((CI, HD), jnp.float32),
            pltpu.VMEM((CQ * DM // 128, 128), jnp.float32),
            pltpu.VMEM((CQ * DM // 128, 128), jnp.float32),
            pltpu.SemaphoreType.DMA,
            pltpu.SemaphoreType.DMA,
            pltpu.SemaphoreType.DMA,
            pltpu.SemaphoreType.DMA,
            pltpu.SemaphoreType.DMA,
            pltpu.SemaphoreType.DMA,
        ],
        compiler_params=pltpu.CompilerParams(use_tc_tiling_on_sc=False),
    )
    return f(table, gidx, gw)


def _proj_body(x_ref, wo_ref, bo_ref, out_ref):
    x = x_ref[...].reshape(-1, DM)
    out_ref[...] = (jnp.dot(x.astype(jnp.bfloat16),
                            wo_ref[...].astype(jnp.bfloat16),
                            preferred_element_type=jnp.float32)
                    + bo_ref[...])


def _proj_kernel(x_ref, wo_ref, bo_ref, out_ref):
    _proj_body(x_ref, wo_ref, bo_ref, out_ref)


def _proj_kernel_alias(x_ref, wo_ref, bo_ref, dest_ref, out_ref):
    del dest_ref
    _proj_body(x_ref, wo_ref, bo_ref, out_ref)


def _out_proj(sampled2d, w_o, b_o1, dest, k):
    bt = 1024
    nt = LQS // bt
    kb = LQS // bt * k
    lb = LQ // bt
    in_specs = [
        pl.BlockSpec((2 * bt, 128), lambda b, t: (b * nt + t, 0)),
        pl.BlockSpec((DM, DM), lambda b, t: (0, 0)),
        pl.BlockSpec((1, DM), lambda b, t: (0, 0)),
    ]
    args = (sampled2d, w_o, b_o1)
    body = _proj_kernel
    aliases = {}
    if dest is not None:
        in_specs.append(pl.BlockSpec(memory_space=pl.ANY))
        args = args + (dest,)
        body = _proj_kernel_alias
        aliases = {3: 0}
    return pl.pallas_call(
        body,
        grid=(N_B, nt),
        in_specs=in_specs,
        out_specs=pl.BlockSpec((bt, DM), lambda b, t: (b * lb + kb + t, 0)),
        out_shape=jax.ShapeDtypeStruct((N_B * LQ, DM), jnp.float32),
        input_output_aliases=aliases,
    )(*args)


def kernel(query, reference_points, input_flatten, input_spatial_shapes,
           input_level_start_index, input_padding_mask, W_v, b_v, W_off,
           b_off, W_attn, b_attn, W_o, b_o):
    maskf = input_padding_mask.astype(jnp.float32).reshape(N_B, 1, LIN)
    value = _value_tables(input_flatten, W_v, b_v.reshape(1, DM), maskf)
    table = value.reshape(N_B * LIN * NH, HD)

    woffx = W_off[:, 0::2]
    woffy = W_off[:, 1::2]
    boffx = b_off[0::2].reshape(1, 32)
    boffy = b_off[1::2].reshape(1, 32)
    rp = reference_points.reshape(N_B, LQ, 4)
    b_attn1 = b_attn.reshape(1, 32)
    b_o1 = b_o.reshape(1, DM)

    dest = None
    for k in range(NSLICE):
        gidx, gw = _index_weights(query, rp, woffx, woffy, boffx, boffy,
                                  W_attn, b_attn1, k)
        sampled = _sc_gather(table, gidx.reshape(-1), gw.reshape(-1))
        dest = _out_proj(sampled.reshape(N_B * LQS * 2, 128), W_o, b_o1,
                         dest, k)
    return dest.reshape(N_B, LQ, DM)
